# baseline (speedup 1.0000x reference)
import functools

import jax
import jax.numpy as jnp
from jax import lax
from jax.experimental import pallas as pl
from jax.experimental.pallas import tpu as pltpu
from jax.experimental.pallas import tpu_sc as plsc

MAXD = 512
D = 128
N = 10000
E = 320000

NC = 2
NS = 16
L = 16

EPT = E // NS
ECH = 80
NCHUNK = EPT // ECH
RB = 80
NBLK = N // RB
BLK_ITERS = (NBLK + NC * NS - 1) // (NC * NS)
HIST = NC * NS * 320

_mesh = plsc.VectorSubcoreMesh(core_axis_name="c", subcore_axis_name="s")


@functools.partial(
    pl.kernel,
    out_type=jax.ShapeDtypeStruct((N, D), jnp.float32),
    mesh=_mesh,
    scratch_types=[
        pltpu.VMEM((NCHUNK, ECH), jnp.int32),
        pltpu.VMEM((ECH,), jnp.int32),
        pltpu.VMEM((HIST // NS,), jnp.int32),
        pltpu.VMEM((RB, D), jnp.float32),
        pltpu.VMEM((RB, D), jnp.float32),
        pltpu.VMEM((RB,), jnp.int32),
        pltpu.VMEM_SHARED((HIST,), jnp.int32),
        pltpu.SemaphoreType.DMA,
    ],
)
def _ce_kernel(esrc, x, z, out, idx_e, ones_v, zer_v, xbuf, zbuf, deg_v,
               hist_sh, sem):
    s = lax.axis_index("s")
    c = lax.axis_index("c")
    wid = s * NC + c

    for k in range(ECH // L):
        ones_v[pl.ds(k * L, L)] = jnp.ones((L,), jnp.int32)
    for k in range((HIST // NS) // L):
        zer_v[pl.ds(k * L, L)] = jnp.zeros((L,), jnp.int32)

    z0 = pl.multiple_of(s * (HIST // NS), 8)
    pltpu.sync_copy(zer_v, hist_sh.at[pl.ds(z0, HIST // NS)])
    plsc.subcore_barrier()

    pltpu.sync_copy(esrc.at[s], idx_e)

    def scatter_body(j, carry):
        pltpu.sync_copy(ones_v, hist_sh.at[idx_e.at[j]], add=True)
        return carry

    lax.fori_loop(0, NCHUNK, scatter_body, 0)
    plsc.subcore_barrier()

    def block_body(j, carry):
        blk = j * (NC * NS) + wid

        @pl.when(blk < NBLK)
        def _():
            base = pl.multiple_of(blk * RB, 8)
            pltpu.sync_copy(x.at[pl.ds(base, RB), :], xbuf)
            pltpu.sync_copy(hist_sh.at[pl.ds(base, RB)], deg_v)
            for k in range(RB // L):
                v = deg_v[pl.ds(k * L, L)]
                deg_v[pl.ds(k * L, L)] = jnp.minimum(v, MAXD - 1)
            pltpu.async_copy(z.at[deg_v], zbuf, sem).wait()

            def add_body(r, acc):
                for cc in range(D // L):
                    xv = xbuf[r, pl.ds(cc * L, L)]
                    zv = zbuf[r, pl.ds(cc * L, L)]
                    xbuf[r, pl.ds(cc * L, L)] = xv + zv
                return acc

            lax.fori_loop(0, RB, add_body, 0)
            pltpu.sync_copy(xbuf, out.at[pl.ds(base, RB), :])

        return carry

    lax.fori_loop(0, BLK_ITERS, block_body, 0)


def kernel(x, edge_index, z):
    esrc = edge_index[0].reshape(NS, NCHUNK, ECH)
    return _ce_kernel(esrc, x, z)


# speedup vs baseline: 1.2435x; 1.2435x over previous
import functools

import jax
import jax.numpy as jnp
from jax import lax
from jax.experimental import pallas as pl
from jax.experimental.pallas import tpu as pltpu
from jax.experimental.pallas import tpu_sc as plsc

MAXD = 512
D = 128
N = 10000
E = 320000

NC = 2
NS = 16
L = 16

EPT = E // NS
ECH = 80
NCHUNK = EPT // ECH
W = 16
RB = 80
NBLK = N // RB
NB = (NBLK + NC * NS - 1) // (NC * NS)
HIST = NC * NS * 320

_mesh = plsc.VectorSubcoreMesh(core_axis_name="c", subcore_axis_name="s")


@functools.partial(
    pl.kernel,
    out_type=jax.ShapeDtypeStruct((N, D), jnp.float32),
    mesh=_mesh,
    scratch_types=[
        pltpu.VMEM((NCHUNK, ECH), jnp.int32),
        pltpu.VMEM((ECH,), jnp.int32),
        pltpu.VMEM((HIST // NS,), jnp.int32),
        pltpu.VMEM((NB, RB, D), jnp.float32),
        pltpu.VMEM((NB, RB, D), jnp.float32),
        pltpu.VMEM((NB, RB), jnp.int32),
        pltpu.VMEM_SHARED((HIST,), jnp.int32),
        pltpu.SemaphoreType.DMA,
        pltpu.SemaphoreType.DMA,
        pltpu.SemaphoreType.DMA((NB,)),
        pltpu.SemaphoreType.DMA((NB,)),
        pltpu.SemaphoreType.DMA((NB,)),
    ],
)
def _ce_kernel(esrc, x, z, out, idx_e, ones_v, zer_v, X, Z, dg, hist_sh,
               semE, semS, semX, semZ, semO):
    s = lax.axis_index("s")
    c = lax.axis_index("c")
    wid = s * NC + c

    edge_cp = pltpu.async_copy(esrc.at[s], idx_e, semE)

    for k in range(ECH // L):
        ones_v[pl.ds(k * L, L)] = jnp.ones((L,), jnp.int32)
    for k in range((HIST // NS) // L):
        zer_v[pl.ds(k * L, L)] = jnp.zeros((L,), jnp.int32)

    z0 = pl.multiple_of(s * (HIST // NS), 8)
    pltpu.sync_copy(zer_v, hist_sh.at[pl.ds(z0, HIST // NS)])
    edge_cp.wait()
    plsc.subcore_barrier()

    def scatter_body(j, carry):
        pltpu.async_copy(ones_v, hist_sh.at[idx_e.at[j]], semS, add=True)

        @pl.when(j >= W)
        def _():
            pltpu.make_async_copy(ones_v, hist_sh.at[idx_e.at[0]], semS).wait()

        return carry

    lax.fori_loop(0, NCHUNK, scatter_body, 0)

    def drain_body(j, carry):
        pltpu.make_async_copy(ones_v, hist_sh.at[idx_e.at[0]], semS).wait()
        return carry

    lax.fori_loop(0, W, drain_body, 0)
    plsc.subcore_barrier()

    def base_of(b):
        return pl.multiple_of((b * (NC * NS) + wid) * RB, 8)

    for b in range(NB):
        @pl.when(b * (NC * NS) + wid < NBLK)
        def _(b=b):
            pltpu.async_copy(x.at[pl.ds(base_of(b), RB), :], X.at[b], semX.at[b])

    for b in range(NB):
        @pl.when(b * (NC * NS) + wid < NBLK)
        def _(b=b):
            dgb = dg.at[b]
            pltpu.sync_copy(hist_sh.at[pl.ds(base_of(b), RB)], dgb)
            for k in range(RB // L):
                v = dgb[pl.ds(k * L, L)]
                dgb[pl.ds(k * L, L)] = jnp.minimum(v, MAXD - 1)
            pltpu.async_copy(z.at[dgb], Z.at[b], semZ.at[b])

    for b in range(NB):
        @pl.when(b * (NC * NS) + wid < NBLK)
        def _(b=b):
            base = base_of(b)
            pltpu.make_async_copy(x.at[pl.ds(base, RB), :], X.at[b],
                                  semX.at[b]).wait()
            pltpu.make_async_copy(z.at[dg.at[b]], Z.at[b], semZ.at[b]).wait()
            Xb = X.at[b]
            Zb = Z.at[b]

            def add_body(r, acc):
                for cc in range(D // L):
                    xv = Xb[r, pl.ds(cc * L, L)]
                    zv = Zb[r, pl.ds(cc * L, L)]
                    Xb[r, pl.ds(cc * L, L)] = xv + zv
                return acc

            lax.fori_loop(0, RB, add_body, 0)
            pltpu.async_copy(X.at[b], out.at[pl.ds(base, RB), :], semO.at[b])

    for b in range(NB):
        @pl.when(b * (NC * NS) + wid < NBLK)
        def _(b=b):
            pltpu.make_async_copy(X.at[b], out.at[pl.ds(base_of(b), RB), :],
                                  semO.at[b]).wait()


def kernel(x, edge_index, z):
    esrc = edge_index[0].reshape(NS, NCHUNK, ECH)
    return _ce_kernel(esrc, x, z)


# speedup vs baseline: 1.3534x; 1.0884x over previous
import functools

import jax
import jax.numpy as jnp
from jax import lax
from jax.experimental import pallas as pl
from jax.experimental.pallas import tpu as pltpu
from jax.experimental.pallas import tpu_sc as plsc

MAXD = 512
D = 128
N = 10000
E = 320000

NC = 2
NS = 16
L = 16

ECH = 80
NCHUNK = E // (NC * NS) // ECH
W = 16
HIST = NC * NS * 320
HSL = HIST // NS

_mesh = plsc.VectorSubcoreMesh(core_axis_name="c", subcore_axis_name="s")


@functools.partial(
    pl.kernel,
    out_type=jax.ShapeDtypeStruct((NC, HIST), jnp.int32),
    mesh=_mesh,
    scratch_types=[
        pltpu.VMEM((NCHUNK, ECH), jnp.int32),
        pltpu.VMEM((ECH,), jnp.int32),
        pltpu.VMEM((HSL,), jnp.int32),
        pltpu.VMEM_SHARED((HIST,), jnp.int32),
        pltpu.SemaphoreType.DMA,
        pltpu.SemaphoreType.DMA,
    ],
)
def _hist_kernel(esrc, out, idx_e, ones_v, stg_v, hist_sh, semE, semS):
    s = lax.axis_index("s")
    c = lax.axis_index("c")

    edge_cp = pltpu.async_copy(esrc.at[c, s], idx_e, semE)

    for k in range(ECH // L):
        ones_v[pl.ds(k * L, L)] = jnp.ones((L,), jnp.int32)
    for k in range(HSL // L):
        stg_v[pl.ds(k * L, L)] = jnp.zeros((L,), jnp.int32)

    h0 = pl.multiple_of(s * HSL, 8)
    pltpu.sync_copy(stg_v, hist_sh.at[pl.ds(h0, HSL)])
    edge_cp.wait()
    plsc.subcore_barrier()

    def scatter_body(j, carry):
        pltpu.async_copy(ones_v, hist_sh.at[idx_e.at[j]], semS, add=True)

        @pl.when(j >= W)
        def _():
            pltpu.make_async_copy(ones_v, hist_sh.at[idx_e.at[0]], semS).wait()

        return carry

    lax.fori_loop(0, NCHUNK, scatter_body, 0)

    def drain_body(j, carry):
        pltpu.make_async_copy(ones_v, hist_sh.at[idx_e.at[0]], semS).wait()
        return carry

    lax.fori_loop(0, W, drain_body, 0)
    plsc.subcore_barrier()

    pltpu.sync_copy(hist_sh.at[pl.ds(h0, HSL)], stg_v)
    pltpu.sync_copy(stg_v, out.at[c, pl.ds(h0, HSL)])


RB = 512
G = (N + RB - 1) // RB


def _tc_body(d0_ref, d1_ref, z_ref, x_ref, out_ref):
    dcol = jnp.minimum(d0_ref[...] + d1_ref[...], MAXD - 1)
    one_hot = (dcol == lax.broadcasted_iota(jnp.int32, (RB, MAXD), 1))
    zsel = lax.dot(one_hot.astype(jnp.float32), z_ref[...],
                   precision=lax.Precision.HIGHEST,
                   preferred_element_type=jnp.float32)
    out_ref[...] = x_ref[...] + zsel


_gather_add = pl.pallas_call(
    _tc_body,
    grid=(G,),
    in_specs=[
        pl.BlockSpec((RB, 1), lambda i: (i, 0)),
        pl.BlockSpec((RB, 1), lambda i: (i, 0)),
        pl.BlockSpec((MAXD, D), lambda i: (0, 0)),
        pl.BlockSpec((RB, D), lambda i: (i, 0)),
    ],
    out_specs=pl.BlockSpec((RB, D), lambda i: (i, 0)),
    out_shape=jax.ShapeDtypeStruct((N, D), jnp.float32),
)


def kernel(x, edge_index, z):
    esrc = edge_index[0].reshape(NC, NS, NCHUNK, ECH)
    hist2 = _hist_kernel(esrc)
    d0 = hist2[0].reshape(HIST, 1)
    d1 = hist2[1].reshape(HIST, 1)
    return _gather_add(d0, d1, z, x)


# speedup vs baseline: 2.0603x; 1.5224x over previous
import functools

import jax
import jax.numpy as jnp
from jax import lax
from jax.experimental import pallas as pl
from jax.experimental.pallas import tpu as pltpu
from jax.experimental.pallas import tpu_sc as plsc

MAXD = 512
D = 128
N = 10000
E = 320000

NC = 2
NS = 16
L = 16

ECH = 80
NCHUNK = E // (NC * NS) // ECH
W = 16
HIST = NC * NS * 320
HSL = HIST // NS

_mesh = plsc.VectorSubcoreMesh(core_axis_name="c", subcore_axis_name="s")


@functools.partial(
    pl.kernel,
    out_type=(
        jax.ShapeDtypeStruct((HIST,), jnp.int32),
        jax.ShapeDtypeStruct((HIST,), jnp.int32),
    ),
    mesh=_mesh,
    scratch_types=[
        pltpu.VMEM((NCHUNK, ECH), jnp.int32),
        pltpu.VMEM((ECH,), jnp.int32),
        pltpu.VMEM((HSL,), jnp.int32),
        pltpu.VMEM_SHARED((HIST,), jnp.int32),
        pltpu.SemaphoreType.DMA,
        pltpu.SemaphoreType.DMA,
    ],
)
def _hist_kernel(esrc, out0, out1, idx_e, ones_v, stg_v, hist_sh, semE, semS):
    s = lax.axis_index("s")
    c = lax.axis_index("c")

    edge_cp = pltpu.async_copy(esrc.at[0, c, s], idx_e, semE)

    for k in range(ECH // L):
        ones_v[pl.ds(k * L, L)] = jnp.ones((L,), jnp.int32)
    for k in range(HSL // L):
        stg_v[pl.ds(k * L, L)] = jnp.zeros((L,), jnp.int32)

    h0 = pl.multiple_of(s * HSL, 8)
    pltpu.sync_copy(stg_v, hist_sh.at[pl.ds(h0, HSL)])
    edge_cp.wait()
    plsc.subcore_barrier()

    def scatter_body(j, carry):
        pltpu.async_copy(ones_v, hist_sh.at[idx_e.at[j]], semS, add=True)

        @pl.when(j >= W)
        def _():
            pltpu.make_async_copy(ones_v, hist_sh.at[idx_e.at[0]], semS).wait()

        return carry

    lax.fori_loop(0, NCHUNK, scatter_body, 0)

    def drain_body(j, carry):
        pltpu.make_async_copy(ones_v, hist_sh.at[idx_e.at[0]], semS).wait()
        return carry

    lax.fori_loop(0, W, drain_body, 0)
    plsc.subcore_barrier()

    pltpu.sync_copy(hist_sh.at[pl.ds(h0, HSL)], stg_v)

    @pl.when(c == 0)
    def _():
        pltpu.sync_copy(stg_v, out0.at[pl.ds(h0, HSL)])

    @pl.when(c == 1)
    def _():
        pltpu.sync_copy(stg_v, out1.at[pl.ds(h0, HSL)])


RB = 512
G = (N + RB - 1) // RB
SB = RB // 128


def _tc_body(d0_ref, d1_ref, zhi_ref, zlo_ref, x_ref, out_ref):
    d = jnp.minimum(d0_ref[0] + d1_ref[0], MAXD - 1)
    rdiv = lax.broadcasted_iota(jnp.int32, (RB, 128), 0) // 128
    t = jnp.zeros((RB, 128), jnp.int32)
    for j in range(SB):
        t = jnp.where(rdiv == j, jnp.broadcast_to(d[j:j + 1, :], (RB, 128)), t)
    lane = lax.broadcasted_iota(jnp.int32, (RB, 128), 1)
    row = lax.broadcasted_iota(jnp.int32, (RB, 128), 0)
    dcol = jnp.sum(jnp.where(lane == row % 128, t, 0), axis=1, keepdims=True)
    one_hot = (dcol == lax.broadcasted_iota(jnp.int32, (RB, MAXD), 1))
    e_bf = one_hot.astype(jnp.bfloat16)
    zsel = (lax.dot(e_bf, zhi_ref[...], preferred_element_type=jnp.float32)
            + lax.dot(e_bf, zlo_ref[...], preferred_element_type=jnp.float32))
    out_ref[...] = x_ref[...] + zsel


_gather_add = pl.pallas_call(
    _tc_body,
    grid=(G,),
    in_specs=[
        pl.BlockSpec((1, SB, 128), lambda i: (i, 0, 0)),
        pl.BlockSpec((1, SB, 128), lambda i: (i, 0, 0)),
        pl.BlockSpec((MAXD, D), lambda i: (0, 0)),
        pl.BlockSpec((MAXD, D), lambda i: (0, 0)),
        pl.BlockSpec((RB, D), lambda i: (i, 0)),
    ],
    out_specs=pl.BlockSpec((RB, D), lambda i: (i, 0)),
    out_shape=jax.ShapeDtypeStruct((N, D), jnp.float32),
)


def kernel(x, edge_index, z):
    esrc = edge_index.reshape(2, NC, NS, NCHUNK, ECH)
    h0, h1 = _hist_kernel(esrc)
    d0 = h0.reshape(G, SB, 128)
    d1 = h1.reshape(G, SB, 128)
    zhi = z.astype(jnp.bfloat16)
    zlo = (z - zhi.astype(jnp.float32)).astype(jnp.bfloat16)
    return _gather_add(d0, d1, zhi, zlo, x)
